# Initial kernel scaffold; baseline (speedup 1.0000x reference)
import jax
import jax.numpy as jnp
from jax.experimental import pallas as pl


def kernel(frame_idx, bbox_idx, pose_adjustment):
    raise NotImplementedError("write your pallas kernel here")


# speedup vs baseline: 2.3262x; 2.3262x over previous
import jax
import jax.numpy as jnp
from jax import lax
from jax.experimental import pallas as pl
from jax.experimental.pallas import tpu as pltpu
from jax.experimental.pallas import tpu_sc as plsc

_NUM_FRAMES = 10000
_NUM_BBOXES = 512
_N = 1048576
_PLANE = _NUM_FRAMES * _NUM_BBOXES

_NC = 2
_NS = 16
_NW = _NC * _NS
_L = 16

_CHUNK = 2048
_DMA_ROWS = 128
_NB = _CHUNK // _DMA_ROWS
_GPB = _DMA_ROWS // _L
_ROWS_PER_TILE = _N // _NW
_CHUNKS_PER_TILE = _ROWS_PER_TILE // _CHUNK
_NBLK = _N // 128

_COS_C = [-1.0 / 87178291200.0, 1.0 / 479001600.0, -1.0 / 3628800.0,
          1.0 / 40320.0, -1.0 / 720.0, 1.0 / 24.0, -1.0 / 2.0, 1.0]
_SINC_C = [-1.0 / 1307674368000.0, 1.0 / 6227020800.0, -1.0 / 39916800.0,
           1.0 / 362880.0, -1.0 / 5040.0, 1.0 / 120.0, -1.0 / 6.0, 1.0]
_OMC_C = [-1.0 / 20922789888000.0, 1.0 / 87178291200.0, -1.0 / 479001600.0,
          1.0 / 3628800.0, -1.0 / 40320.0, 1.0 / 720.0, -1.0 / 24.0, 0.5]


def _horner(coeffs, u):
    acc = jnp.full((_L,), coeffs[0], jnp.float32)
    for c in coeffs[1:]:
        acc = acc * u + jnp.float32(c)
    return acc


def _sc_body(idx_hbm, table_hbm, out_hbm, idx_v, rows_v, out_v, sem):
    wid = lax.axis_index("s") * _NC + lax.axis_index("c")
    iota = lax.broadcasted_iota(jnp.int32, (_L,), 0)

    def chunk_body(cc, carry):
        row0 = pl.multiple_of(wid * _ROWS_PER_TILE + cc * _CHUNK, _CHUNK)
        idx_row0 = pl.multiple_of(row0 // 128, _NB)
        pltpu.sync_copy(idx_hbm.at[pl.ds(idx_row0, _NB), :], idx_v)

        def fire(b):
            handles = []
            for c in range(6):
                handles.append(pltpu.async_copy(
                    table_hbm.at[pl.ds(c * _PLANE, _PLANE)].at[idx_v.at[b]],
                    rows_v.at[c, pl.ds(b * _DMA_ROWS, _DMA_ROWS)],
                    sem))
            return handles

        def compute(b):
            def group_body(g, carry2):
                rid = b * _DMA_ROWS + g * _L + iota
                comp = [rows_v[c, pl.ds(b * _DMA_ROWS + g * _L, _L)]
                        for c in range(6)]
                tx, ty, tz, ax, ay, az = comp
                u = ax * ax + ay * ay + az * az
                cos = _horner(_COS_C, u)
                sinc = _horner(_SINC_C, u)
                omc = _horner(_OMC_C, u)
                s0 = sinc * ax
                s1 = sinc * ay
                s2 = sinc * az
                ox = omc * ax
                oy = omc * ay
                oz = omc * az
                vals = [
                    [ox * ax + cos, ox * ay - s2, ox * az + s1, tx],
                    [oy * ax + s2, oy * ay + cos, oy * az - s0, ty],
                    [oz * ax - s1, oz * ay + s0, oz * az + cos, tz],
                ]
                r4 = lax.shift_left(lax.shift_right_logical(rid, 7),
                                    jnp.int32(2))
                col = lax.bitwise_and(rid, jnp.int32(127))
                for i in range(3):
                    for j in range(4):
                        plsc.store_scatter(
                            out_v,
                            [jnp.full((_L,), i, jnp.int32), r4 + jnp.int32(j),
                             col],
                            vals[i][j])
                return carry2

            lax.fori_loop(0, _GPB, group_body, 0)

        handles = fire(0)
        for b in range(_NB):
            nxt = fire(b + 1) if b + 1 < _NB else []
            for h in handles:
                h.wait()
            compute(b)
            handles = nxt

        out_row0 = pl.multiple_of((row0 // 128) * 4, _CHUNK // 32)
        for i in range(3):
            pltpu.sync_copy(
                out_v.at[i],
                out_hbm.at[pl.ds(i * (_NBLK * 4) + out_row0, _NB * 4), :])
        return carry

    lax.fori_loop(0, _CHUNKS_PER_TILE, chunk_body, 0)


def kernel(frame_idx, bbox_idx, pose_adjustment):
    flat = (lax.shift_left(lax.shift_right_logical(frame_idx, 3), 12)
            + lax.shift_left(lax.shift_right_logical(bbox_idx, 7), 10)
            + lax.shift_left(jnp.bitwise_and(frame_idx, 7), 7)
            + jnp.bitwise_and(bbox_idx, 127))
    table = (jnp.transpose(pose_adjustment, (2, 0, 1))
             .reshape(6, 1250, 8, 4, 128)
             .transpose(0, 1, 3, 2, 4)
             .reshape(6 * _PLANE))
    mesh = plsc.VectorSubcoreMesh(
        core_axis_name="c", subcore_axis_name="s",
        num_cores=_NC, num_subcores=_NS)
    out = pl.kernel(
        _sc_body,
        out_type=jax.ShapeDtypeStruct((3 * _NBLK * 4, 128), jnp.float32),
        mesh=mesh,
        compiler_params=pltpu.CompilerParams(needs_layout_passes=False),
        scratch_types=[
            pltpu.VMEM((_NB, 128), jnp.int32),
            pltpu.VMEM((6, _CHUNK), jnp.float32),
            pltpu.VMEM((3, _NB * 4, 128), jnp.float32),
            pltpu.SemaphoreType.DMA,
        ],
    )(flat.reshape(_NBLK, 128), table)
    return (out.reshape(3, _NBLK, 4, 128)
               .transpose(1, 3, 0, 2)
               .reshape(_N, 3, 4))


# speedup vs baseline: 3.0591x; 1.3151x over previous
import jax
import jax.numpy as jnp
from jax import lax
from jax.experimental import pallas as pl
from jax.experimental.pallas import tpu as pltpu
from jax.experimental.pallas import tpu_sc as plsc

_NUM_FRAMES = 10000
_NUM_BBOXES = 512
_N = 1048576
_PLANE = _NUM_FRAMES * _NUM_BBOXES

_NC = 2
_NS = 16
_NW = _NC * _NS
_L = 16

_BPT = (_N // 128) // _NW
_NBLK = _N // 128

_COS_C = [-1.0 / 87178291200.0, 1.0 / 479001600.0, -1.0 / 3628800.0,
          1.0 / 40320.0, -1.0 / 720.0, 1.0 / 24.0, -1.0 / 2.0, 1.0]
_SINC_C = [-1.0 / 1307674368000.0, 1.0 / 6227020800.0, -1.0 / 39916800.0,
           1.0 / 362880.0, -1.0 / 5040.0, 1.0 / 120.0, -1.0 / 6.0, 1.0]
_OMC_C = [-1.0 / 20922789888000.0, 1.0 / 87178291200.0, -1.0 / 479001600.0,
          1.0 / 3628800.0, -1.0 / 40320.0, 1.0 / 720.0, -1.0 / 24.0, 0.5]


def _horner(coeffs, u):
    acc = jnp.full((_L,), coeffs[0], jnp.float32)
    for c in coeffs[1:]:
        acc = acc * u + jnp.float32(c)
    return acc


def _sc_body(idx_hbm, table_hbm, out_hbm,
             idx_v, rows_v, out_v,
             sem_g0, sem_g1, sem_o0, sem_o1):
    wid = lax.axis_index("s") * _NC + lax.axis_index("c")
    sem_g = (sem_g0, sem_g1)
    sem_o = (sem_o0, sem_o1)
    bt0 = wid * _BPT

    pltpu.sync_copy(
        idx_hbm.at[pl.ds(pl.multiple_of(wid * _BPT, _BPT), _BPT), :], idx_v)

    def gather_descs(t, par):
        return [pltpu.make_async_copy(
                    table_hbm.at[pl.ds(c * _PLANE, _PLANE)].at[idx_v.at[t]],
                    rows_v.at[par * 6 + c], sem_g[par])
                for c in range(6)]

    def fire_gather(t, par):
        for d in gather_descs(t, par):
            d.start()

    def drain_gather(t, par):
        for d in gather_descs(t, par):
            d.wait()

    def compute(par, slot, half):
        for g in range(8):
            comp = [rows_v[par * 6 + c, pl.ds(g * _L, _L)]
                    for c in range(6)]
            tx, ty, tz, ax, ay, az = comp
            u = ax * ax + ay * ay + az * az
            cos = _horner(_COS_C, u)
            sinc = _horner(_SINC_C, u)
            omc = _horner(_OMC_C, u)
            s0 = sinc * ax
            s1 = sinc * ay
            s2 = sinc * az
            ox = omc * ax
            oy = omc * ay
            oz = omc * az
            vals = [
                [ox * ax + cos, ox * ay - s2, ox * az + s1, tx],
                [oy * ax + s2, oy * ay + cos, oy * az - s0, ty],
                [oz * ax - s1, oz * ay + s0, oz * az + cos, tz],
            ]
            for i in range(3):
                for j in range(4):
                    out_v[(slot * 3 + i) * 8 + half * 4 + j,
                          pl.ds(g * _L, _L)] = vals[i][j]

    def out_descs(t, slot):
        row0 = pl.multiple_of((bt0 + t - 1) * 4, 8)
        return [pltpu.make_async_copy(
                    out_v.at[pl.ds((slot * 3 + i) * 8, 8), :],
                    out_hbm.at[pl.ds(i * (_NBLK * 4) + row0, 8), :],
                    sem_o[slot])
                for i in range(3)]

    def flush_out(t, slot):
        for d in out_descs(t, slot):
            d.start()

    def drain_out(t, slot):
        for d in out_descs(t, slot):
            d.wait()

    fire_gather(0, 0)

    def step(s, carry):
        for k in range(4):
            t = s * 4 + k
            par = k & 1
            slot = k >> 1
            half = k & 1
            if k == 3:
                @pl.when(s < (_BPT // 4) - 1)
                def _():
                    fire_gather(t + 1, 0)
            else:
                fire_gather(t + 1, (k + 1) & 1)
            drain_gather(t, par)
            if k in (0, 2):
                @pl.when(s > 0)
                def _():
                    drain_out((s - 1) * 4 + k + 1, slot)
            compute(par, slot, half)
            if k in (1, 3):
                flush_out(t, slot)
        return carry

    lax.fori_loop(0, _BPT // 4, step, 0)
    last = (_BPT // 4 - 1) * 4
    drain_out(last + 1, 0)
    drain_out(last + 3, 1)


def kernel(frame_idx, bbox_idx, pose_adjustment):
    flat = (lax.shift_left(lax.shift_right_logical(frame_idx, 3), 12)
            + lax.shift_left(lax.shift_right_logical(bbox_idx, 7), 10)
            + lax.shift_left(jnp.bitwise_and(frame_idx, 7), 7)
            + jnp.bitwise_and(bbox_idx, 127))
    table = (jnp.transpose(pose_adjustment, (2, 0, 1))
             .reshape(6, 1250, 8, 4, 128)
             .transpose(0, 1, 3, 2, 4)
             .reshape(6 * _PLANE))
    mesh = plsc.VectorSubcoreMesh(
        core_axis_name="c", subcore_axis_name="s",
        num_cores=_NC, num_subcores=_NS)
    out = pl.kernel(
        _sc_body,
        out_type=jax.ShapeDtypeStruct((3 * _NBLK * 4, 128), jnp.float32),
        mesh=mesh,
        compiler_params=pltpu.CompilerParams(needs_layout_passes=False),
        scratch_types=[
            pltpu.VMEM((_BPT, 128), jnp.int32),
            pltpu.VMEM((12, 128), jnp.float32),
            pltpu.VMEM((48, 128), jnp.float32),
            pltpu.SemaphoreType.DMA,
            pltpu.SemaphoreType.DMA,
            pltpu.SemaphoreType.DMA,
            pltpu.SemaphoreType.DMA,
        ],
    )(flat.reshape(_NBLK, 128), table)
    return (out.reshape(3, _NBLK, 4, 128)
               .transpose(1, 3, 0, 2)
               .reshape(_N, 3, 4))
